# baseline (speedup 1.0000x reference)
import jax
import jax.numpy as jnp
from jax import lax
from jax.experimental import pallas as pl
from jax.experimental.pallas import tpu as pltpu
from jax.experimental.pallas import tpu_sc as plsc

_K0 = 0x03D7B32D
_K1 = 0xADD083F4

_LANES = 16


def _threefry_bits_u32(x1):
    ks = (
        jnp.uint32(_K0),
        jnp.uint32(_K1),
        jnp.uint32(_K0 ^ _K1 ^ 0x1BD11BDA),
    )
    rots = ((13, 15, 26, 6), (17, 29, 16, 24))
    x0 = jnp.zeros(x1.shape, jnp.uint32) + ks[0]
    x1 = x1 + ks[1]
    for g in range(5):
        for r in rots[g % 2]:
            x0 = x0 + x1
            x1 = (x1 << jnp.uint32(r)) | (x1 >> jnp.uint32(32 - r))
            x1 = x1 ^ x0
        x0 = x0 + ks[(g + 1) % 3]
        x1 = x1 + ks[(g + 2) % 3] + jnp.uint32(g + 1)
    return x0 ^ x1


def _make_sc_kernel(b_dim, c_dim, h_dim, w_dim):
    hp = h_dim // 2
    wp = w_dim // 2
    l_dim = hp * wp
    n_jv = wp // _LANES
    tasks = b_dim * hp
    mesh = plsc.VectorSubcoreMesh(core_axis_name="c", subcore_axis_name="s")
    n_workers = mesh.num_cores * mesh.num_subcores
    tpw = tasks // n_workers
    assert tpw * n_workers == tasks and n_jv * _LANES == wp
    assert tpw % 2 == 0

    ch = c_dim // 2
    supers = b_dim * (hp // 2)
    spw = supers // n_workers
    assert spw * n_workers == supers

    def body(x_hbm, o_hbm, xb0, xb1, ob0, ob1, si0, si1, so0, so1):
        cid = lax.axis_index("c")
        sid = lax.axis_index("s")
        wid = sid * mesh.num_cores + cid
        s0 = wid * spw
        lane = lax.iota(jnp.int32, _LANES)
        xbufs = (xb0, xb1)
        obufs = (ob0, ob1)
        sis = (si0, si1)
        sos = (so0, so1)

        def rowflat_of(s):
            b = s // (hp // 2)
            ip = s % (hp // 2)
            rfs = []
            for rr in range(2):
                for jv in range(n_jv):
                    j = jv * _LANES + lane
                    f = (b * l_dim + (2 * ip + rr) * wp + j).astype(jnp.uint32)
                    k = (_threefry_bits_u32(f) & jnp.uint32(3)).astype(jnp.int32)
                    rfs.append((2 * rr + (k >> 1), 2 * j + (k & 1)))
            return tuple(rfs)

        def in_desc(s, h):
            b = s // (hp // 2)
            ip = s % (hp // 2)
            return (
                x_hbm.at[b, pl.ds(ch * h, ch), pl.ds(4 * ip, 4), :],
                xbufs[h],
                sis[h],
            )

        def out_desc(s, h):
            b = s // (hp // 2)
            ip = s % (hp // 2)
            return (
                obufs[h],
                o_hbm.at[b, pl.ds(ch * h, ch), pl.ds(2 * ip, 2), :],
                sos[h],
            )

        pltpu.async_copy(*in_desc(s0, 0))
        rf_first = rowflat_of(s0)

        def super_body(p, rf_carry):
            s = s0 + p
            rf = rf_carry
            rf_next = rf
            for h in range(2):
                if h == 0:
                    pltpu.async_copy(*in_desc(s, 1))
                else:

                    @pl.when(s + 1 < s0 + spw)
                    def _():
                        pltpu.async_copy(*in_desc(s + 1, 0))

                    rf_next = rowflat_of(s + 1)

                pltpu.make_async_copy(*in_desc(s, h)).wait()

                @pl.when(p >= 1)
                def _():
                    pltpu.make_async_copy(*out_desc(s - 1, h)).wait()

                xb = xbufs[h]
                ob = obufs[h]

                def ch_body(c, rf=rf, xb=xb, ob=ob):
                    cs = jnp.zeros((_LANES,), jnp.int32) + c
                    for g in range(2 * n_jv):
                        dr, col = rf[g]
                        vals = plsc.load_gather(xb, [cs, dr, col])
                        ob[c, g // n_jv, pl.ds((g % n_jv) * _LANES, _LANES)] = vals

                plsc.parallel_loop(0, ch, 1, unroll=2)(ch_body)
                pltpu.async_copy(*out_desc(s, h))
            return rf_next

        lax.fori_loop(0, spw, super_body, rf_first, unroll=False)
        pltpu.make_async_copy(*out_desc(s0 + spw - 1, 0)).wait()
        pltpu.make_async_copy(*out_desc(s0 + spw - 1, 1)).wait()

    ker = pl.kernel(
        body,
        out_type=jax.ShapeDtypeStruct((b_dim, c_dim, hp, wp), jnp.float32),
        mesh=mesh,
        compiler_params=pltpu.CompilerParams(
            use_tc_tiling_on_sc=True, needs_layout_passes=False
        ),
        scratch_types=[
            pltpu.VMEM((c_dim // 2, 4, w_dim), jnp.float32),
            pltpu.VMEM((c_dim // 2, 4, w_dim), jnp.float32),
            pltpu.VMEM((c_dim // 2, 2, wp), jnp.float32),
            pltpu.VMEM((c_dim // 2, 2, wp), jnp.float32),
            pltpu.SemaphoreType.DMA,
            pltpu.SemaphoreType.DMA,
            pltpu.SemaphoreType.DMA,
            pltpu.SemaphoreType.DMA,
        ],
    )

    return ker


@jax.jit
def kernel(x):
    b_dim, c_dim, h_dim, w_dim = x.shape
    return _make_sc_kernel(b_dim, c_dim, h_dim, w_dim)(x)


# speedup vs baseline: 1.0023x; 1.0023x over previous
import jax
import jax.numpy as jnp
from jax import lax
from jax.experimental import pallas as pl
from jax.experimental.pallas import tpu as pltpu
from jax.experimental.pallas import tpu_sc as plsc

_K0 = 0x03D7B32D
_K1 = 0xADD083F4

_LANES = 16


def _threefry_bits_u32(x1):
    ks = (
        jnp.uint32(_K0),
        jnp.uint32(_K1),
        jnp.uint32(_K0 ^ _K1 ^ 0x1BD11BDA),
    )
    rots = ((13, 15, 26, 6), (17, 29, 16, 24))
    x0 = jnp.zeros(x1.shape, jnp.uint32) + ks[0]
    x1 = x1 + ks[1]
    for g in range(5):
        for r in rots[g % 2]:
            x0 = x0 + x1
            x1 = (x1 << jnp.uint32(r)) | (x1 >> jnp.uint32(32 - r))
            x1 = x1 ^ x0
        x0 = x0 + ks[(g + 1) % 3]
        x1 = x1 + ks[(g + 2) % 3] + jnp.uint32(g + 1)
    return x0 ^ x1


def _make_sc_kernel(b_dim, c_dim, h_dim, w_dim):
    hp = h_dim // 2
    wp = w_dim // 2
    l_dim = hp * wp
    n_jv = wp // _LANES
    tasks = b_dim * hp
    mesh = plsc.VectorSubcoreMesh(core_axis_name="c", subcore_axis_name="s")
    n_workers = mesh.num_cores * mesh.num_subcores
    tpw = tasks // n_workers
    assert tpw * n_workers == tasks and n_jv * _LANES == wp
    assert tpw % 2 == 0

    ch = c_dim // 2
    supers = b_dim * (hp // 2)
    spw = supers // n_workers
    assert spw * n_workers == supers

    def body(x_hbm, o_hbm, xb0, xb1, ob0, ob1, si0, si1, so0, so1):
        cid = lax.axis_index("c")
        sid = lax.axis_index("s")
        wid = sid * mesh.num_cores + cid
        s0 = wid * spw
        lane = lax.iota(jnp.int32, _LANES)
        xbufs = (xb0, xb1)
        obufs = (ob0, ob1)
        sis = (si0, si1)
        sos = (so0, so1)

        def rowflat_of(s):
            b = s // (hp // 2)
            ip = s % (hp // 2)
            rfs = []
            for rr in range(2):
                for jv in range(n_jv):
                    j = jv * _LANES + lane
                    f = (b * l_dim + (2 * ip + rr) * wp + j).astype(jnp.uint32)
                    k = (_threefry_bits_u32(f) & jnp.uint32(3)).astype(jnp.int32)
                    rfs.append((2 * rr + (k >> 1), 2 * j + (k & 1)))
            return tuple(rfs)

        def in_desc(s, h):
            b = s // (hp // 2)
            ip = s % (hp // 2)
            return (
                x_hbm.at[b, pl.ds(ch * h, ch), pl.ds(4 * ip, 4), :],
                xbufs[h],
                sis[h],
            )

        def out_desc(s, h):
            b = s // (hp // 2)
            ip = s % (hp // 2)
            return (
                obufs[h],
                o_hbm.at[b, pl.ds(ch * h, ch), pl.ds(2 * ip, 2), :],
                sos[h],
            )

        pltpu.async_copy(*in_desc(s0, 0))
        rf_first = rowflat_of(s0)

        def super_body(p, rf_carry):
            s = s0 + p
            rf = rf_carry
            rf_next = rf
            for h in range(2):
                if h == 0:
                    pltpu.async_copy(*in_desc(s, 1))
                else:

                    @pl.when(s + 1 < s0 + spw)
                    def _():
                        pltpu.async_copy(*in_desc(s + 1, 0))

                    rf_next = rowflat_of(s + 1)

                pltpu.make_async_copy(*in_desc(s, h)).wait()

                @pl.when(p >= 1)
                def _():
                    pltpu.make_async_copy(*out_desc(s - 1, h)).wait()

                xb = xbufs[h]
                ob = obufs[h]

                def ch_body(c, rf=rf, xb=xb, ob=ob):
                    cs = jnp.zeros((_LANES,), jnp.int32) + c
                    for g in range(2 * n_jv):
                        dr, col = rf[g]
                        vals = plsc.load_gather(xb, [cs, dr, col])
                        ob[c, g // n_jv, pl.ds((g % n_jv) * _LANES, _LANES)] = vals

                plsc.parallel_loop(0, ch, 1, unroll=4)(ch_body)
                pltpu.async_copy(*out_desc(s, h))
            return rf_next

        lax.fori_loop(0, spw, super_body, rf_first, unroll=False)
        pltpu.make_async_copy(*out_desc(s0 + spw - 1, 0)).wait()
        pltpu.make_async_copy(*out_desc(s0 + spw - 1, 1)).wait()

    ker = pl.kernel(
        body,
        out_type=jax.ShapeDtypeStruct((b_dim, c_dim, hp, wp), jnp.float32),
        mesh=mesh,
        compiler_params=pltpu.CompilerParams(
            use_tc_tiling_on_sc=True, needs_layout_passes=False
        ),
        scratch_types=[
            pltpu.VMEM((c_dim // 2, 4, w_dim), jnp.float32),
            pltpu.VMEM((c_dim // 2, 4, w_dim), jnp.float32),
            pltpu.VMEM((c_dim // 2, 2, wp), jnp.float32),
            pltpu.VMEM((c_dim // 2, 2, wp), jnp.float32),
            pltpu.SemaphoreType.DMA,
            pltpu.SemaphoreType.DMA,
            pltpu.SemaphoreType.DMA,
            pltpu.SemaphoreType.DMA,
        ],
    )

    return ker


@jax.jit
def kernel(x):
    b_dim, c_dim, h_dim, w_dim = x.shape
    return _make_sc_kernel(b_dim, c_dim, h_dim, w_dim)(x)


# speedup vs baseline: 1.0324x; 1.0300x over previous
import jax
import jax.numpy as jnp
from jax import lax
from jax.experimental import pallas as pl
from jax.experimental.pallas import tpu as pltpu
from jax.experimental.pallas import tpu_sc as plsc

_K0 = 0x03D7B32D
_K1 = 0xADD083F4

_LANES = 16


def _threefry_bits_u32(x1):
    ks = (
        jnp.uint32(_K0),
        jnp.uint32(_K1),
        jnp.uint32(_K0 ^ _K1 ^ 0x1BD11BDA),
    )
    rots = ((13, 15, 26, 6), (17, 29, 16, 24))
    x0 = jnp.zeros(x1.shape, jnp.uint32) + ks[0]
    x1 = x1 + ks[1]
    for g in range(5):
        for r in rots[g % 2]:
            x0 = x0 + x1
            x1 = (x1 << jnp.uint32(r)) | (x1 >> jnp.uint32(32 - r))
            x1 = x1 ^ x0
        x0 = x0 + ks[(g + 1) % 3]
        x1 = x1 + ks[(g + 2) % 3] + jnp.uint32(g + 1)
    return x0 ^ x1


def _make_sc_kernel(b_dim, c_dim, h_dim, w_dim):
    hp = h_dim // 2
    wp = w_dim // 2
    l_dim = hp * wp
    n_jv = wp // _LANES
    tasks = b_dim * hp
    mesh = plsc.VectorSubcoreMesh(core_axis_name="c", subcore_axis_name="s")
    n_workers = mesh.num_cores * mesh.num_subcores
    tpw = tasks // n_workers
    assert tpw * n_workers == tasks and n_jv * _LANES == wp
    assert tpw % 2 == 0

    def body(x_hbm, o_hbm, xb0, xb1, ob0, ob1, si0, si1, so0, so1):
        cid = lax.axis_index("c")
        sid = lax.axis_index("s")
        wid = sid * mesh.num_cores + cid
        t0 = wid * tpw
        lane = lax.iota(jnp.int32, _LANES)
        xbufs = (xb0, xb1)
        obufs = (ob0, ob1)
        sis = (si0, si1)
        sos = (so0, so1)

        def rowflat_of(t):
            b = t // hp
            i = t % hp
            rfs = []
            for jv in range(n_jv):
                j = jv * _LANES + lane
                f = (b * l_dim + i * wp + j).astype(jnp.uint32)
                k = (_threefry_bits_u32(f) & jnp.uint32(3)).astype(jnp.int32)
                rfs.append((k >> 1, 2 * j + (k & 1)))
            return tuple(rfs)

        def in_desc(t, hb):
            b = t // hp
            i = t % hp
            return (x_hbm.at[b, :, pl.ds(2 * i, 2), :], xbufs[hb], sis[hb])

        def out_desc(t, hb):
            b = t // hp
            i = t % hp
            return (obufs[hb], o_hbm.at[b, :, i, :], sos[hb])

        pltpu.async_copy(*in_desc(t0, 0))
        rf_first = rowflat_of(t0)

        def pair_body(p, rf_carry):
            rf = rf_carry
            for hb in range(2):
                t = t0 + 2 * p + hb
                tn = t + 1

                @pl.when(tn < t0 + tpw)
                def _():
                    pltpu.async_copy(*in_desc(tn, 1 - hb))

                rf_next = rowflat_of(tn)
                pltpu.make_async_copy(*in_desc(t, hb)).wait()

                @pl.when(t >= t0 + 2)
                def _():
                    pltpu.make_async_copy(*out_desc(t - 2, hb)).wait()

                xb = xbufs[hb]
                ob = obufs[hb]

                def ch_body(c, rf=rf, xb=xb, ob=ob):
                    cs = jnp.zeros((_LANES,), jnp.int32) + c
                    for jv in range(n_jv):
                        di, col = rf[jv]
                        vals = plsc.load_gather(xb, [cs, di, col])
                        ob[c, pl.ds(jv * _LANES, _LANES)] = vals

                plsc.parallel_loop(0, c_dim, 1, unroll=4)(ch_body)
                pltpu.async_copy(*out_desc(t, hb))
                rf = rf_next
            return rf

        lax.fori_loop(0, tpw // 2, pair_body, rf_first, unroll=False)
        pltpu.make_async_copy(*out_desc(t0 + tpw - 2, 0)).wait()
        pltpu.make_async_copy(*out_desc(t0 + tpw - 1, 1)).wait()

    ker = pl.kernel(
        body,
        out_type=jax.ShapeDtypeStruct((b_dim, c_dim, hp, wp), jnp.float32),
        mesh=mesh,
        compiler_params=pltpu.CompilerParams(
            use_tc_tiling_on_sc=True, needs_layout_passes=False
        ),
        scratch_types=[
            pltpu.VMEM((c_dim, 2, w_dim), jnp.float32),
            pltpu.VMEM((c_dim, 2, w_dim), jnp.float32),
            pltpu.VMEM((c_dim, wp), jnp.float32),
            pltpu.VMEM((c_dim, wp), jnp.float32),
            pltpu.SemaphoreType.DMA,
            pltpu.SemaphoreType.DMA,
            pltpu.SemaphoreType.DMA,
            pltpu.SemaphoreType.DMA,
        ],
    )

    return ker


@jax.jit
def kernel(x):
    b_dim, c_dim, h_dim, w_dim = x.shape
    return _make_sc_kernel(b_dim, c_dim, h_dim, w_dim)(x)
